# baseline (speedup 1.0000x reference)
import functools

import jax
import jax.numpy as jnp
from jax import lax
from jax.experimental import pallas as pl
from jax.experimental.pallas import tpu as pltpu
from jax.experimental.pallas import tpu_sc as plsc

_C = 3
_N = 64 * 512 * 512
_NW = 32
_CHUNK = _N // _NW
_ZB = 65536
_NZ = _CHUNK // _ZB


def _sc_body(tensor_hbm, keep_hbm, out_hbm, keep_v, zbuf, sem):
    wid = lax.axis_index("s") * 2 + lax.axis_index("c")
    base = wid * _CHUNK

    pltpu.sync_copy(keep_hbm, keep_v)

    zv = jnp.zeros((16,), jnp.float32)

    def _zero(i, carry):
        zbuf[pl.ds(i * 16, 16)] = zv
        return carry

    lax.fori_loop(0, _ZB // 16, _zero, 0)

    kvec = keep_v[...]
    for c in range(_C):
        keep_c = kvec[c]

        @pl.when(keep_c > 0)
        def _copy(c=c, base=base):
            pltpu.sync_copy(
                tensor_hbm.at[pl.ds(c * _N + base, _CHUNK)],
                out_hbm.at[pl.ds(c * _N + base, _CHUNK)],
            )

        @pl.when(keep_c == 0)
        def _zero_out(c=c, base=base):
            def _one(j, carry):
                pltpu.sync_copy(
                    zbuf, out_hbm.at[pl.ds(c * _N + base + j * _ZB, _ZB)]
                )
                return carry

            lax.fori_loop(0, _NZ, _one, 0)


@functools.partial(
    pl.kernel,
    mesh=plsc.VectorSubcoreMesh(core_axis_name="c", subcore_axis_name="s"),
    out_type=jax.ShapeDtypeStruct((_C * _N,), jnp.float32),
    scratch_types=[
        pltpu.VMEM((16,), jnp.int32),
        pltpu.VMEM((_ZB,), jnp.float32),
        pltpu.SemaphoreType.DMA,
    ],
)
def _sc_kernel(tensor_hbm, keep_hbm, out_hbm, keep_v, zbuf, sem):
    _sc_body(tensor_hbm, keep_hbm, out_hbm, keep_v, zbuf, sem)


def kernel(tensor, skip_prob):
    u = jax.random.uniform(jax.random.key(42), (3,), dtype=jnp.float32)
    keep = (u > skip_prob).astype(jnp.int32)
    keep16 = jnp.pad(keep, (0, 16 - _C))
    flat = tensor.reshape(_C * _N)
    out = _sc_kernel(flat, keep16)
    return out.reshape(tensor.shape)


# speedup vs baseline: 8.3834x; 8.3834x over previous
import jax
import jax.numpy as jnp
from jax.experimental import pallas as pl
from jax.experimental.pallas import tpu as pltpu

_C = 3
_ROWS = 16384
_LANES = 1024
_BS = 512
_NB = _ROWS // _BS


def _body(keep_ref, in_ref, out_ref):
    c = pl.program_id(0)
    keep_c = keep_ref[c]

    @pl.when(keep_c > 0)
    def _copy():
        out_ref[...] = in_ref[...]

    @pl.when(keep_c == 0)
    def _zero():
        out_ref[...] = jnp.zeros_like(out_ref)


def _in_map(c, b, keep_ref):
    return c, jnp.where(keep_ref[c] > 0, b, 0), 0


def _out_map(c, b, keep_ref):
    return c, b, 0


def kernel(tensor, skip_prob):
    u = jax.random.uniform(jax.random.key(42), (3,), dtype=jnp.float32)
    keep = (u > skip_prob).astype(jnp.int32)
    t3 = tensor.reshape(_C, _ROWS, _LANES)
    out = pl.pallas_call(
        _body,
        grid_spec=pltpu.PrefetchScalarGridSpec(
            num_scalar_prefetch=1,
            grid=(_C, _NB),
            in_specs=[pl.BlockSpec((1, _BS, _LANES), _in_map)],
            out_specs=pl.BlockSpec((1, _BS, _LANES), _out_map),
        ),
        out_shape=jax.ShapeDtypeStruct((_C, _ROWS, _LANES), jnp.float32),
    )(keep, t3)
    return out.reshape(tensor.shape)


# speedup vs baseline: 8.5550x; 1.0205x over previous
import jax
import jax.numpy as jnp
from jax.experimental import pallas as pl
from jax.experimental.pallas import tpu as pltpu

_C = 3
_ROWS = 16384
_LANES = 1024
_BS = 2048
_NB = _ROWS // _BS


def _body(keep_ref, in_ref, out_ref):
    c = pl.program_id(0)
    keep_c = keep_ref[c]

    @pl.when(keep_c > 0)
    def _copy():
        out_ref[...] = in_ref[...]

    @pl.when(keep_c == 0)
    def _zero():
        out_ref[...] = jnp.zeros_like(out_ref)


def _in_map(c, b, keep_ref):
    return c, jnp.where(keep_ref[c] > 0, b, 0), 0


def _out_map(c, b, keep_ref):
    return c, b, 0


def kernel(tensor, skip_prob):
    u = jax.random.uniform(jax.random.key(42), (3,), dtype=jnp.float32)
    keep = (u > skip_prob).astype(jnp.int32)
    t3 = tensor.reshape(_C, _ROWS, _LANES)
    out = pl.pallas_call(
        _body,
        grid_spec=pltpu.PrefetchScalarGridSpec(
            num_scalar_prefetch=1,
            grid=(_C, _NB),
            in_specs=[pl.BlockSpec((1, _BS, _LANES), _in_map)],
            out_specs=pl.BlockSpec((1, _BS, _LANES), _out_map),
        ),
        out_shape=jax.ShapeDtypeStruct((_C, _ROWS, _LANES), jnp.float32),
    )(keep, t3)
    return out.reshape(tensor.shape)
